# baseline (speedup 1.0000x reference)
import functools

import jax
import jax.numpy as jnp
from jax import lax
from jax.experimental import pallas as pl
from jax.experimental.pallas import tpu as pltpu
from jax.experimental.pallas import tpu_sc as plsc

VOCAB_SIZE = 100000
D = 32
B = 1024

V_TILE = 2048
N_V = (VOCAB_SIZE + V_TILE - 1) // V_TILE

_NC = 2
_NS = 16
_NW = _NC * _NS
_B_PER_W = B // _NW


def _gather_body(idx_hbm, table_hbm, out_hbm, idx_v, rows_v, sem):
    wid = lax.axis_index("s") * _NC + lax.axis_index("c")
    base = wid * _B_PER_W
    pltpu.sync_copy(idx_hbm.at[pl.ds(base, _B_PER_W)], idx_v)
    pltpu.async_copy(table_hbm.at[idx_v], rows_v, sem).wait()
    pltpu.sync_copy(rows_v, out_hbm.at[pl.ds(base, _B_PER_W)])


def _sc_gather(tokens, table):
    f = pl.kernel(
        _gather_body,
        out_type=jax.ShapeDtypeStruct((B, D), jnp.float32),
        mesh=plsc.VectorSubcoreMesh(core_axis_name="c", subcore_axis_name="s"),
        scratch_types=[
            pltpu.VMEM((_B_PER_W,), jnp.int32),
            pltpu.VMEM((_B_PER_W, D), jnp.float32),
            pltpu.SemaphoreType.DMA,
        ],
        compiler_params=pltpu.CompilerParams(use_tc_tiling_on_sc=False),
    )
    return f(tokens, table)


def _softmax_body(emb_ref, w_ref, b_ref, out_ref, m_ref, s_ref):
    p = pl.program_id(0)
    v = pl.program_id(1)

    @pl.when(jnp.logical_and(p == 0, v == 0))
    def _init():
        m_ref[...] = jnp.full_like(m_ref, -jnp.inf)
        s_ref[...] = jnp.zeros_like(s_ref)

    logits = lax.dot_general(
        emb_ref[...], w_ref[...],
        dimension_numbers=(((1,), (1,)), ((), ())),
        preferred_element_type=jnp.float32,
    ) + b_ref[...]

    lane = lax.broadcasted_iota(jnp.int32, logits.shape, 1)
    valid = lane < (VOCAB_SIZE - v * V_TILE)
    logits = jnp.where(valid, logits, -1e30)

    @pl.when(p == 0)
    def _pass0():
        m_old = m_ref[...]
        m_tile = jnp.max(logits, axis=1, keepdims=True)
        m_new = jnp.maximum(m_old, m_tile)
        s_ref[...] = s_ref[...] * jnp.exp(m_old - m_new) + jnp.sum(
            jnp.exp(logits - m_new), axis=1, keepdims=True)
        m_ref[...] = m_new

    @pl.when(p == 1)
    def _pass1():
        out_ref[...] = jnp.exp(logits - m_ref[...]) * (1.0 / s_ref[...])


def _tc_softmax(emb, W, b):
    return pl.pallas_call(
        _softmax_body,
        grid=(2, N_V),
        in_specs=[
            pl.BlockSpec((B, D), lambda p, v: (0, 0)),
            pl.BlockSpec((V_TILE, D), lambda p, v: (v, 0)),
            pl.BlockSpec((1, V_TILE), lambda p, v: (0, v)),
        ],
        out_specs=pl.BlockSpec((B, V_TILE), lambda p, v: (0, v * p)),
        out_shape=jax.ShapeDtypeStruct((B, VOCAB_SIZE), jnp.float32),
        scratch_shapes=[
            pltpu.VMEM((B, 1), jnp.float32),
            pltpu.VMEM((B, 1), jnp.float32),
        ],
        compiler_params=pltpu.CompilerParams(
            dimension_semantics=("arbitrary", "arbitrary")),
    )(emb, W, b.reshape(1, VOCAB_SIZE))


def kernel(tokenized_input, emb_table, W, b):
    emb = _sc_gather(tokenized_input.astype(jnp.int32), emb_table)
    return _tc_softmax(emb, W, b)


# speedup vs baseline: 3.1066x; 3.1066x over previous
import jax
import jax.numpy as jnp
from jax import lax
from jax.experimental import pallas as pl
from jax.experimental.pallas import tpu as pltpu
from jax.experimental.pallas import tpu_sc as plsc

VOCAB_SIZE = 100000
D = 32
B = 1024

V_TILE = 2048
N_V = (VOCAB_SIZE + V_TILE - 1) // V_TILE
V_PAD = N_V * V_TILE

_NC = 2
_NS = 16
_NW = _NC * _NS
_B_PER_W = B // _NW


def _gather_body(idx_hbm, tbl_t_hbm, out_hbm, idx_v, cols_v, sem):
    wid = lax.axis_index("s") * _NC + lax.axis_index("c")
    base = wid * _B_PER_W
    pltpu.sync_copy(idx_hbm.at[pl.ds(base, _B_PER_W)], idx_v)
    for k0 in range(0, D, 16):
        copies = [
            pltpu.async_copy(tbl_t_hbm.at[k].at[idx_v], cols_v.at[k], sem)
            for k in range(k0, k0 + 16)
        ]
        for c in copies:
            c.wait()
    pltpu.sync_copy(cols_v, out_hbm.at[:, pl.ds(base, _B_PER_W)])


def _sc_gather(tokens, table_t):
    f = pl.kernel(
        _gather_body,
        out_type=jax.ShapeDtypeStruct((D, B), jnp.float32),
        mesh=plsc.VectorSubcoreMesh(core_axis_name="c", subcore_axis_name="s"),
        scratch_types=[
            pltpu.VMEM((_B_PER_W,), jnp.int32),
            pltpu.VMEM((D, _B_PER_W), jnp.float32),
            pltpu.SemaphoreType.DMA,
        ],
        compiler_params=pltpu.CompilerParams(use_tc_tiling_on_sc=False),
    )
    return f(tokens, table_t)


def _mm(w_ref, e_ref):
    return lax.dot_general(
        w_ref[...], e_ref[...],
        dimension_numbers=(((0,), (0,)), ((), ())),
        preferred_element_type=jnp.float32,
    )


_N_CHUNK = 4
_CHUNK = V_TILE // _N_CHUNK


def _softmax_body(w_ref, e_ref, out_ref, s_ref, c_ref):
    p = pl.program_id(0)
    v = pl.program_id(1)

    @pl.when(p == 0)
    def _accum():
        @pl.when(v == 0)
        def _init():
            s_ref[...] = jnp.zeros_like(s_ref)

        total = jnp.zeros((1, B), jnp.float32)
        for c in range(_N_CHUNK):
            l2c = lax.dot_general(
                w_ref[:, c * _CHUNK:(c + 1) * _CHUNK], e_ref[...],
                dimension_numbers=(((0,), (0,)), ((), ())),
                preferred_element_type=jnp.float32,
            )
            total = total + jnp.sum(jnp.exp2(l2c), axis=0, keepdims=True)
        s_ref[...] = s_ref[...] + total

        @pl.when(v == N_V - 1)
        def _finalize():
            c_ref[...] = jnp.log2(s_ref[...])

    @pl.when(p == 1)
    def _emit():
        l2 = _mm(w_ref, e_ref)
        out_ref[...] = jnp.exp2(l2 - c_ref[...])


_LOG2E = 1.4426950408889634


def _tc_softmax(emb_t, W, b):
    wt_aug = jnp.concatenate([
        jnp.pad(W.T, ((0, 0), (0, V_PAD - VOCAB_SIZE))),
        jnp.pad(b[None, :] * _LOG2E, ((0, 0), (0, V_PAD - VOCAB_SIZE)),
                constant_values=-1e30),
    ], axis=0).astype(jnp.bfloat16)
    e_aug = jnp.concatenate(
        [emb_t * _LOG2E, jnp.ones((1, B), jnp.float32)],
        axis=0).astype(jnp.bfloat16)
    out_t = pl.pallas_call(
        _softmax_body,
        grid=(2, N_V),
        in_specs=[
            pl.BlockSpec((D + 1, V_TILE), lambda p, v: (0, v)),
            pl.BlockSpec((D + 1, B), lambda p, v: (0, 0)),
        ],
        out_specs=pl.BlockSpec((V_TILE, B), lambda p, v: (v * p, 0)),
        out_shape=jax.ShapeDtypeStruct((VOCAB_SIZE, B), jnp.float32),
        scratch_shapes=[
            pltpu.VMEM((1, B), jnp.float32),
            pltpu.VMEM((1, B), jnp.float32),
        ],
        compiler_params=pltpu.CompilerParams(
            dimension_semantics=("arbitrary", "arbitrary")),
    )(wt_aug, e_aug)
    return out_t.T


def kernel(tokenized_input, emb_table, W, b):
    emb_t = _sc_gather(tokenized_input.astype(jnp.int32), emb_table.T)
    return _tc_softmax(emb_t, W, b)
